# baseline (speedup 1.0000x reference)
import functools

import jax
import jax.numpy as jnp
from jax import lax
from jax.experimental import pallas as pl
from jax.experimental.pallas import tpu as pltpu
from jax.experimental.pallas import tpu_sc as plsc

N_NODES = 10000
N_EDGES = 320000
D_IN = 128
D_HID = 128
D_OUT = 64
NUM_GRAPHS = 16

NW = 32
CK = 128
GROUPS = 10
CHUNKS = 8 * GROUPS
HALF = CHUNKS // 2
E_TILE = CK * CHUNKS
E_PAD = E_TILE * NW
N_SC = 10112
ROWS_PER_TILE = N_SC // 16

_mesh = plsc.VectorSubcoreMesh(core_axis_name="c", subcore_axis_name="s")
_sc_params = pltpu.CompilerParams(needs_layout_passes=False)


def _sc_stream_body(tab_hbm, gidx_hbm, sidx_hbm, z_hbm, out,
                    gbuf, sbuf, rb0, rb1, acc_sh, gs0, gs1, ss0, ss1):
    cid = lax.axis_index("c")
    sid = lax.axis_index("s")
    wid = cid * 16 + sid
    r0 = sid * ROWS_PER_TILE

    pltpu.sync_copy(z_hbm.at[pl.ds(r0, ROWS_PER_TILE)],
                    acc_sh.at[pl.ds(r0, ROWS_PER_TILE)])
    plsc.subcore_barrier()

    bufs = (rb0, rb1)
    gsems = (gs0, gs1)
    ssems = (ss0, ss1)
    for h in range(CHUNKS // HALF):
        pltpu.sync_copy(gidx_hbm.at[wid, pl.ds(h * HALF, HALF)], gbuf)
        pltpu.sync_copy(sidx_hbm.at[wid, pl.ds(h * HALF, HALF)], sbuf)
        pend_g = pltpu.async_copy(tab_hbm.at[gbuf.at[0]], bufs[0], gsems[0])
        pend_s = [None, None]
        for r in range(HALF):
            b = r % 2
            nb = (r + 1) % 2
            if r + 1 < HALF:
                if pend_s[nb] is not None:
                    pend_s[nb].wait()
                nxt = pltpu.async_copy(tab_hbm.at[gbuf.at[r + 1]],
                                       bufs[nb], gsems[nb])
            pend_g.wait()
            pend_s[b] = pltpu.async_copy(bufs[b], acc_sh.at[sbuf.at[r]],
                                         ssems[b], add=True)
            if r + 1 < HALF:
                pend_g = nxt
        pend_s[(HALF - 1) % 2].wait()
        pend_s[HALF % 2].wait()

    plsc.subcore_barrier()
    pltpu.sync_copy(acc_sh.at[pl.ds(r0, ROWS_PER_TILE)],
                    out.at[cid, pl.ds(r0, ROWS_PER_TILE)])


_sc_scratch = [
    pltpu.VMEM((HALF, CK), jnp.int32),
    pltpu.VMEM((HALF, CK), jnp.int32),
    pltpu.VMEM((CK, D_IN), jnp.float32),
    pltpu.VMEM((CK, D_IN), jnp.float32),
    pltpu.VMEM_SHARED((N_SC, D_IN), jnp.float32),
    pltpu.SemaphoreType.DMA,
    pltpu.SemaphoreType.DMA,
    pltpu.SemaphoreType.DMA,
    pltpu.SemaphoreType.DMA,
]


@functools.partial(
    pl.kernel,
    out_type=(jax.ShapeDtypeStruct((2, N_SC, D_IN), jnp.float32),
              jax.ShapeDtypeStruct((2, N_SC, D_IN), jnp.float32)),
    mesh=_mesh,
    scratch_types=_sc_scratch,
    compiler_params=_sc_params,
)
def _sc_passes(x_hbm, b1h_hbm, src_hbm, dst_hbm, z_hbm, A_out, C_out, *scr):
    _sc_stream_body(x_hbm, src_hbm, dst_hbm, z_hbm, A_out, *scr)
    _sc_stream_body(b1h_hbm, dst_hbm, src_hbm, z_hbm, C_out, *scr)


_BLK = 1264
_NBLK = N_SC // _BLK


def _tc_body(x_ref, A0_ref, A1_ref, C0_ref, C1_ref, b_ref,
             Wrel1_ref, Wroot1_ref, b1_ref, Wrel2_ref, Wroot2_ref, b2_ref,
             Wout_ref, bout_ref, out_ref, accS, accN, accC):
    i = pl.program_id(0)
    f32 = jnp.float32

    A = A0_ref[...] + A1_ref[...]
    h1 = jnp.maximum(
        jnp.dot(A, Wrel1_ref[...], preferred_element_type=f32)
        + jnp.dot(x_ref[...], Wroot1_ref[...], preferred_element_type=f32)
        + b1_ref[...], 0.0)

    rows = i * _BLK + lax.broadcasted_iota(jnp.int32, (_BLK, 1), 0)
    valid = rows < N_NODES
    h1 = jnp.where(valid, h1, 0.0)
    C = jnp.where(valid, C0_ref[...] + C1_ref[...], 0.0)
    giota = lax.broadcasted_iota(jnp.int32, (_BLK, NUM_GRAPHS), 1)
    onehot = jnp.where(valid & (b_ref[...] == giota), 1.0, 0.0)

    dn = (((0,), (0,)), ((), ()))
    S_part = lax.dot_general(C, h1, dn, preferred_element_type=f32)
    N_part = lax.dot_general(onehot, h1, dn, preferred_element_type=f32)
    cnt_part = lax.dot_general(onehot, jnp.ones((_BLK, 1), f32), dn,
                               preferred_element_type=f32)

    @pl.when(i == 0)
    def _():
        accS[...] = jnp.zeros_like(accS)
        accN[...] = jnp.zeros_like(accN)
        accC[...] = jnp.zeros_like(accC)

    accS[...] += S_part
    accN[...] += N_part
    accC[...] += cnt_part

    @pl.when(i == _NBLK - 1)
    def _():
        cnt = accC[...]
        pooled_sum = (
            jnp.dot(accS[0:NUM_GRAPHS, :], Wrel2_ref[...],
                    preferred_element_type=f32)
            + jnp.dot(accN[...], Wroot2_ref[...], preferred_element_type=f32)
            + cnt * b2_ref[...])
        pooled = pooled_sum / jnp.maximum(cnt, 1.0)
        out_ref[...] = (jnp.dot(pooled, Wout_ref[...],
                                preferred_element_type=f32) + bout_ref[...])


def _tc_pass(x_pad, A0, A1, C0, C1, batch_col,
             W_rel1, W_root1, b1, W_rel2, W_root2, b2, W_out, b_out):
    full = lambda shape: pl.BlockSpec(shape, lambda i: (0, 0))
    blk = lambda shape: pl.BlockSpec(shape, lambda i: (i, 0))
    return pl.pallas_call(
        _tc_body,
        grid=(_NBLK,),
        in_specs=[
            blk((_BLK, D_IN)),
            blk((_BLK, D_IN)),
            blk((_BLK, D_IN)),
            blk((_BLK, D_IN)),
            blk((_BLK, D_IN)),
            blk((_BLK, 1)),
            full((D_IN, D_HID)),
            full((D_IN, D_HID)),
            full((1, D_HID)),
            full((D_HID, D_HID)),
            full((D_HID, D_HID)),
            full((1, D_HID)),
            full((D_HID, D_OUT)),
            full((1, D_OUT)),
        ],
        out_specs=pl.BlockSpec((NUM_GRAPHS, D_OUT), lambda i: (0, 0)),
        out_shape=jax.ShapeDtypeStruct((NUM_GRAPHS, D_OUT), jnp.float32),
        scratch_shapes=[
            pltpu.VMEM((D_IN, D_HID), jnp.float32),
            pltpu.VMEM((NUM_GRAPHS, D_HID), jnp.float32),
            pltpu.VMEM((NUM_GRAPHS, 1), jnp.float32),
        ],
        compiler_params=pltpu.CompilerParams(
            dimension_semantics=("arbitrary",)),
    )(x_pad, A0, A1, C0, C1, batch_col,
      W_rel1, W_root1, b1, W_rel2, W_root2, b2, W_out, b_out)


def kernel(x, edge_index, batch_idx, W_rel1, W_root1, b1,
           W_rel2, W_root2, b2, W_out, b_out):
    src = edge_index[0].astype(jnp.int32)
    dst = edge_index[1].astype(jnp.int32)
    pad_e = E_PAD - N_EDGES
    srcp = jnp.concatenate(
        [src, jnp.full((pad_e,), N_NODES, jnp.int32)]).reshape(NW, CHUNKS, CK)
    dstp = jnp.concatenate(
        [dst, jnp.full((pad_e,), N_NODES, jnp.int32)]).reshape(NW, CHUNKS, CK)
    batch_ext = jnp.concatenate(
        [batch_idx.astype(jnp.int32),
         jnp.zeros((N_SC - N_NODES,), jnp.int32)])
    b1h = jax.nn.one_hot(batch_ext, D_IN, dtype=jnp.float32)
    x_pad = jnp.concatenate(
        [x, jnp.zeros((N_SC - N_NODES, D_IN), jnp.float32)])
    zA = jnp.zeros((N_SC, D_IN), jnp.float32)

    A_parts, C_parts = _sc_passes(x_pad, b1h, srcp, dstp, zA)

    return _tc_pass(
        x_pad, A_parts[0], A_parts[1], C_parts[0], C_parts[1],
        batch_ext.reshape(N_SC, 1),
        W_rel1, W_root1, b1.reshape(1, D_HID),
        W_rel2, W_root2, b2.reshape(1, D_HID),
        W_out, b_out.reshape(1, D_OUT))
